# baseline (device time: 21653 ns/iter reference)
import jax
import jax.numpy as jnp
from jax import lax
from jax.experimental import pallas as pl
from jax.experimental.pallas import tpu as pltpu

N_DEV = 4
B = 2
SQ = 128
SKV_SHARD = 128
SKV = N_DEV * SKV_SHARD
D = 512
HQ = 8
HKV = 2
DH = 64
HPD = HQ // N_DEV
HCOLS = HPD * DH


def kernel(x, Wq, Wo, K_ext, V_ext):
    my_out = lax.axis_index("i")
    x2d = x.reshape(B * SQ, D)
    wq_my = lax.dynamic_slice(Wq, (0, my_out * HCOLS), (D, HCOLS))
    kt = jnp.transpose(K_ext, (2, 0, 1, 3))
    vt = jnp.transpose(V_ext, (2, 0, 1, 3))

    def body(x_ref, wq_ref, wo_ref, k_ref, v_ref, out_ref,
             kfull, vfull, qs, attn_my, attn_buf,
             ksend, krecv, vsend, vrecv, asend, arecv, locsem):
        my = lax.axis_index("i")
        my_kvh = my // 2

        barrier_sem = pltpu.get_barrier_semaphore()
        for d in range(1, N_DEV):
            pl.semaphore_signal(
                barrier_sem, inc=1,
                device_id=((my + d) % N_DEV,),
                device_id_type=pl.DeviceIdType.MESH,
            )
        pl.semaphore_wait(barrier_sem, N_DEV - 1)

        ck = pltpu.make_async_copy(
            k_ref.at[my_kvh], kfull.at[:, 0:SKV_SHARD, :], locsem.at[0])
        cv = pltpu.make_async_copy(
            v_ref.at[my_kvh], vfull.at[:, 0:SKV_SHARD, :], locsem.at[1])
        ck.start()
        cv.start()

        p1 = []
        for d in (2, 1, 3):
            tgt = (my + d) % N_DEV
            kvh_t = tgt // 2
            sl = slice(d * SKV_SHARD, (d + 1) * SKV_SHARD)
            rk = pltpu.make_async_remote_copy(
                src_ref=k_ref.at[kvh_t], dst_ref=kfull.at[:, sl, :],
                send_sem=ksend.at[d - 1], recv_sem=krecv.at[d - 1],
                device_id=(tgt,), device_id_type=pl.DeviceIdType.MESH,
            )
            rv = pltpu.make_async_remote_copy(
                src_ref=v_ref.at[kvh_t], dst_ref=vfull.at[:, sl, :],
                send_sem=vsend.at[d - 1], recv_sem=vrecv.at[d - 1],
                device_id=(tgt,), device_id_type=pl.DeviceIdType.MESH,
            )
            rk.start()
            rv.start()
            p1.append((rk, rv))

        qmy = jnp.dot(x_ref[...], wq_ref[...],
                      preferred_element_type=jnp.float32) * 0.125
        for b in range(B):
            for hh in range(HPD):
                qs[b, hh * SQ:(hh + 1) * SQ, :] = (
                    qmy[b * SQ:(b + 1) * SQ, hh * DH:(hh + 1) * DH])

        ck.wait()
        cv.wait()
        for d in (1, 3, 2):
            sl = slice(d * SKV_SHARD, (d + 1) * SKV_SHARD)
            pltpu.make_async_remote_copy(
                src_ref=k_ref.at[0], dst_ref=kfull.at[:, sl, :],
                send_sem=ksend.at[d - 1], recv_sem=krecv.at[d - 1],
                device_id=(my,), device_id_type=pl.DeviceIdType.MESH,
            ).wait_recv()
            pltpu.make_async_remote_copy(
                src_ref=v_ref.at[0], dst_ref=vfull.at[:, sl, :],
                send_sem=vsend.at[d - 1], recv_sem=vrecv.at[d - 1],
                device_id=(my,), device_id_type=pl.DeviceIdType.MESH,
            ).wait_recv()

        for b in range(B):
            qb = qs[b]
            s = lax.dot_general(
                qb, kfull[b], (((1,), (1,)), ((), ())),
                preferred_element_type=jnp.float32)
            p = jnp.exp(s)
            linv = 1.0 / jnp.sum(p, axis=1, keepdims=True)
            o = jnp.dot(p, vfull[b],
                        preferred_element_type=jnp.float32) * linv
            for hh in range(HPD):
                attn_my[b * SQ:(b + 1) * SQ, hh * DH:(hh + 1) * DH] = (
                    o[hh * SQ:(hh + 1) * SQ, :])

        ca = pltpu.make_async_copy(attn_my, attn_buf.at[my], locsem.at[2])
        ca.start()
        p3 = []
        for d in (2, 1, 3):
            tgt = (my + d) % N_DEV
            ra = pltpu.make_async_remote_copy(
                src_ref=attn_my, dst_ref=attn_buf.at[my],
                send_sem=asend.at[d - 1], recv_sem=arecv.at[d - 1],
                device_id=(tgt,), device_id_type=pl.DeviceIdType.MESH,
            )
            ra.start()
            p3.append(ra)
        ca.wait()
        for d in (1, 3, 2):
            src_dev = (my - d) % N_DEV
            pltpu.make_async_remote_copy(
                src_ref=attn_my, dst_ref=attn_buf.at[src_dev],
                send_sem=asend.at[d - 1], recv_sem=arecv.at[d - 1],
                device_id=(my,), device_id_type=pl.DeviceIdType.MESH,
            ).wait_recv()

        acc = jnp.dot(attn_buf[0], wo_ref[0:HCOLS, :],
                      preferred_element_type=jnp.float32)
        for c in range(1, N_DEV):
            acc = acc + jnp.dot(
                attn_buf[c], wo_ref[c * HCOLS:(c + 1) * HCOLS, :],
                preferred_element_type=jnp.float32)
        out_ref[...] = acc

        for rk, rv in p1:
            rk.wait_send()
            rv.wait_send()
        for ra in p3:
            ra.wait_send()

    out2d = pl.pallas_call(
        body,
        out_shape=jax.ShapeDtypeStruct((B * SQ, D), jnp.float32),
        in_specs=[pl.BlockSpec(memory_space=pltpu.VMEM)] * 5,
        out_specs=pl.BlockSpec(memory_space=pltpu.VMEM),
        scratch_shapes=[
            pltpu.VMEM((B, SKV, DH), jnp.float32),
            pltpu.VMEM((B, SKV, DH), jnp.float32),
            pltpu.VMEM((B, HPD * SQ, DH), jnp.float32),
            pltpu.VMEM((B * SQ, HCOLS), jnp.float32),
            pltpu.VMEM((N_DEV, B * SQ, HCOLS), jnp.float32),
            pltpu.SemaphoreType.DMA((N_DEV - 1,)),
            pltpu.SemaphoreType.DMA((N_DEV - 1,)),
            pltpu.SemaphoreType.DMA((N_DEV - 1,)),
            pltpu.SemaphoreType.DMA((N_DEV - 1,)),
            pltpu.SemaphoreType.DMA((N_DEV - 1,)),
            pltpu.SemaphoreType.DMA((N_DEV - 1,)),
            pltpu.SemaphoreType.DMA((3,)),
        ],
        compiler_params=pltpu.CompilerParams(collective_id=0),
    )(x2d, wq_my, Wo, kt, vt)
    return out2d.reshape(B, SQ, D)


# device time: 15193 ns/iter; 1.4252x vs baseline; 1.4252x over previous
import jax
import jax.numpy as jnp
from jax import lax
from jax.experimental import pallas as pl
from jax.experimental.pallas import tpu as pltpu

N_DEV = 4
B = 2
SQ = 128
SKV_SHARD = 128
SKV = N_DEV * SKV_SHARD
D = 512
HQ = 8
HKV = 2
DH = 64
HPD = HQ // N_DEV
HCOLS = HPD * DH


def kernel(x, Wq, Wo, K_ext, V_ext):
    my_out = lax.axis_index("i")
    x2d = x.reshape(B * SQ, D)
    wq_my = lax.dynamic_slice(Wq, (0, my_out * HCOLS), (D, HCOLS))
    kt = jnp.transpose(K_ext, (2, 0, 3, 1))
    vt = jnp.transpose(V_ext, (2, 0, 1, 3))

    def body(x_ref, wq_ref, wo_ref, k_ref, v_ref, out_ref,
             kbf, vbf, wobf, kfull, vfull, qs, attn_my, attn_buf,
             ksend, krecv, vsend, vrecv, asend, arecv, locsem):
        my = lax.axis_index("i")
        my_kvh = my // 2

        barrier_sem = pltpu.get_barrier_semaphore()
        for d in range(1, N_DEV):
            pl.semaphore_signal(
                barrier_sem, inc=1,
                device_id=((my + d) % N_DEV,),
                device_id_type=pl.DeviceIdType.MESH,
            )
        kbf[...] = k_ref[...].reshape(HKV * B, DH, SKV_SHARD).astype(
            jnp.bfloat16)
        vbf[...] = v_ref[...].reshape(HKV * B, SKV_SHARD, DH).astype(
            jnp.bfloat16)
        pl.semaphore_wait(barrier_sem, N_DEV - 1)

        ck = []
        cv = []
        for b in range(B):
            c1 = pltpu.make_async_copy(
                kbf.at[my_kvh * B + b], kfull.at[b, :, 0:SKV_SHARD],
                locsem.at[b])
            c2 = pltpu.make_async_copy(
                vbf.at[my_kvh * B + b], vfull.at[b, 0:SKV_SHARD, :],
                locsem.at[B + b])
            c1.start()
            c2.start()
            ck.append(c1)
            cv.append(c2)

        p1 = []
        for b in range(B):
            for d in (2, 1, 3):
                tgt = (my + d) % N_DEV
                kvh_t = tgt // 2
                sl = slice(d * SKV_SHARD, (d + 1) * SKV_SHARD)
                rk = pltpu.make_async_remote_copy(
                    src_ref=kbf.at[kvh_t * B + b], dst_ref=kfull.at[b, :, sl],
                    send_sem=ksend.at[b, d - 1], recv_sem=krecv.at[b, d - 1],
                    device_id=(tgt,), device_id_type=pl.DeviceIdType.MESH,
                )
                rk.start()
                p1.append(rk)
        for b in range(B):
            for d in (2, 1, 3):
                tgt = (my + d) % N_DEV
                kvh_t = tgt // 2
                sl = slice(d * SKV_SHARD, (d + 1) * SKV_SHARD)
                rv = pltpu.make_async_remote_copy(
                    src_ref=vbf.at[kvh_t * B + b], dst_ref=vfull.at[b, sl, :],
                    send_sem=vsend.at[b, d - 1], recv_sem=vrecv.at[b, d - 1],
                    device_id=(tgt,), device_id_type=pl.DeviceIdType.MESH,
                )
                rv.start()
                p1.append(rv)

        qmy = jnp.dot(x_ref[...], wq_ref[...],
                      preferred_element_type=jnp.float32) * (
        0.125 * 1.4426950408889634)
        for b in range(B):
            for hh in range(HPD):
                qs[b, hh * SQ:(hh + 1) * SQ, :] = (
                    qmy[b * SQ:(b + 1) * SQ, hh * DH:(hh + 1) * DH]
                ).astype(jnp.bfloat16)
        wobf[...] = wo_ref[...].astype(jnp.bfloat16)

        pbf = []
        lin = []
        for b in range(B):
            ck[b].wait()
            for d in (1, 3, 2):
                sl = slice(d * SKV_SHARD, (d + 1) * SKV_SHARD)
                pltpu.make_async_remote_copy(
                    src_ref=kbf.at[0], dst_ref=kfull.at[b, :, sl],
                    send_sem=ksend.at[b, d - 1], recv_sem=krecv.at[b, d - 1],
                    device_id=(my,), device_id_type=pl.DeviceIdType.MESH,
                ).wait_recv()
            qb = qs[b]
            s = lax.dot_general(
                qb, kfull[b], (((1,), (0,)), ((), ())),
                preferred_element_type=jnp.float32)
            p = jnp.exp2(s.astype(jnp.bfloat16))
            lin.append(1.0 / jnp.sum(p, axis=1, keepdims=True,
                                     dtype=jnp.float32))
            pbf.append(p)

        p3 = []
        calocal = []
        for b in range(B):
            cv[b].wait()
            for d in (1, 3, 2):
                sl = slice(d * SKV_SHARD, (d + 1) * SKV_SHARD)
                pltpu.make_async_remote_copy(
                    src_ref=vbf.at[0], dst_ref=vfull.at[b, sl, :],
                    send_sem=vsend.at[b, d - 1], recv_sem=vrecv.at[b, d - 1],
                    device_id=(my,), device_id_type=pl.DeviceIdType.MESH,
                ).wait_recv()
            o = jnp.dot(pbf[b], vfull[b],
                        preferred_element_type=jnp.float32) * lin[b]
            for hh in range(HPD):
                attn_my[b, :, hh * DH:(hh + 1) * DH] = (
                    o[hh * SQ:(hh + 1) * SQ, :]).astype(jnp.bfloat16)
            myblk = attn_buf.at[b, :, pl.ds(my * HCOLS, HCOLS)]
            ca = pltpu.make_async_copy(
                attn_my.at[b], myblk, locsem.at[2 * B + b])
            ca.start()
            calocal.append(ca)
            for d in (2, 1, 3):
                tgt = (my + d) % N_DEV
                ra = pltpu.make_async_remote_copy(
                    src_ref=attn_my.at[b], dst_ref=myblk,
                    send_sem=asend.at[b, d - 1], recv_sem=arecv.at[b, d - 1],
                    device_id=(tgt,), device_id_type=pl.DeviceIdType.MESH,
                )
                ra.start()
                p3.append(ra)

        for b in range(B):
            calocal[b].wait()
            for d in (1, 3, 2):
                src_dev = (my - d) % N_DEV
                pltpu.make_async_remote_copy(
                    src_ref=attn_my.at[b],
                    dst_ref=attn_buf.at[b, :, pl.ds(src_dev * HCOLS, HCOLS)],
                    send_sem=asend.at[b, d - 1], recv_sem=arecv.at[b, d - 1],
                    device_id=(my,), device_id_type=pl.DeviceIdType.MESH,
                ).wait_recv()
            out_ref[b * SQ:(b + 1) * SQ, :] = jnp.dot(
                attn_buf[b], wobf[...],
                preferred_element_type=jnp.float32)

        for r in p1:
            r.wait_send()
        for ra in p3:
            ra.wait_send()

    out2d = pl.pallas_call(
        body,
        out_shape=jax.ShapeDtypeStruct((B * SQ, D), jnp.float32),
        in_specs=[pl.BlockSpec(memory_space=pltpu.VMEM)] * 5,
        out_specs=pl.BlockSpec(memory_space=pltpu.VMEM),
        scratch_shapes=[
            pltpu.VMEM((HKV * B, DH, SKV_SHARD), jnp.bfloat16),
            pltpu.VMEM((HKV * B, SKV_SHARD, DH), jnp.bfloat16),
            pltpu.VMEM((D, D), jnp.bfloat16),
            pltpu.VMEM((B, DH, SKV), jnp.bfloat16),
            pltpu.VMEM((B, SKV, DH), jnp.bfloat16),
            pltpu.VMEM((B, HPD * SQ, DH), jnp.bfloat16),
            pltpu.VMEM((B, SQ, HCOLS), jnp.bfloat16),
            pltpu.VMEM((B, SQ, N_DEV * HCOLS), jnp.bfloat16),
            pltpu.SemaphoreType.DMA((B, N_DEV - 1)),
            pltpu.SemaphoreType.DMA((B, N_DEV - 1)),
            pltpu.SemaphoreType.DMA((B, N_DEV - 1)),
            pltpu.SemaphoreType.DMA((B, N_DEV - 1)),
            pltpu.SemaphoreType.DMA((B, N_DEV - 1)),
            pltpu.SemaphoreType.DMA((B, N_DEV - 1)),
            pltpu.SemaphoreType.DMA((2 * B + B,)),
        ],
        compiler_params=pltpu.CompilerParams(collective_id=0),
    )(x2d, wq_my, Wo, kt, vt)
    return out2d.reshape(B, SQ, D)
